# baseline (speedup 1.0000x reference)
import jax
import jax.numpy as jnp
from jax import lax
from jax.experimental import pallas as pl
from jax.experimental.pallas import tpu as pltpu
from jax.experimental.pallas import tpu_sc as plsc

BATCH = 16384
SEQ = 20
EMBED_DIM = 32
NC = 2
NS = 16
NW = NC * NS

B_PER_W = BATCH // NW
IDX_PER_W = B_PER_W * SEQ
IDX_MINOR = 128
IDX_ROWS = IDX_PER_W // IDX_MINOR
CHUNK_SAMPLES = 8
CHUNK_ROWS = CHUNK_SAMPLES * SEQ
GROUPS_PER_CHUNK = CHUNK_ROWS // 16
NCHUNK = B_PER_W // CHUNK_SAMPLES
INV_SEQ = 1.0 / SEQ


def _tree_sum(vals):
  while len(vals) > 1:
    nxt = [vals[i] + vals[i + 1] for i in range(0, len(vals) - 1, 2)]
    if len(vals) % 2:
      nxt.append(vals[-1])
    vals = nxt
  return vals[0]


def _body(idx_hbm, table_hbm, out_hbm,
          idx_v, rows0, rows1, rows2, rows3, out0, out1,
          sem0, sem1, sem2, sem3, osem0, osem1):
  wid = lax.axis_index("s") * NC + lax.axis_index("c")
  base = wid * B_PER_W

  pltpu.sync_copy(idx_hbm.at[wid], idx_v)

  rows = (rows0, rows1, rows2, rows3)
  sems = (sem0, sem1, sem2, sem3)
  outs = (out0, out1)
  osems = (osem0, osem1)

  def fire(bc, buf, sem):
    def grp(g, _):
      gg = bc * GROUPS_PER_CHUNK + g
      vec = idx_v[gg >> 3, pl.ds((gg & 7) * 16, 16)]
      for lane in range(16):
        pltpu.async_copy(
            table_hbm.at[vec[lane]],
            buf.at[g * 16 + lane],
            sem,
        )
      return 0
    lax.fori_loop(0, GROUPS_PER_CHUNK, grp, 0)

  def drain(buf, sem):
    def grp(g, _):
      pltpu.make_async_copy(
          table_hbm.at[pl.ds(0, 16)],
          buf.at[pl.ds(g * 16, 16)],
          sem,
      ).wait()
      return 0
    lax.fori_loop(0, GROUPS_PER_CHUNK, grp, 0)

  def combine(buf, out_v):
    @plsc.parallel_loop(0, CHUNK_SAMPLES, unroll=2)
    def _(i):
      r0 = i * SEQ
      lo = _tree_sum([buf[r0 + s, pl.ds(0, 16)] for s in range(SEQ)])
      hi = _tree_sum([buf[r0 + s, pl.ds(16, 16)] for s in range(SEQ)])
      out_v[i, pl.ds(0, 16)] = lo * INV_SEQ
      out_v[i, pl.ds(16, 16)] = hi * INV_SEQ

  def out_slice(bc):
    return out_hbm.at[pl.ds(base + bc * CHUNK_SAMPLES, CHUNK_SAMPLES)]

  for c in range(3):
    fire(c, rows[c], sems[c])

  def chunk_pair(g, _):
    for b in range(4):
      bc = 4 * g + b
      nb = (b + 3) % 4

      @pl.when(bc + 3 < NCHUNK)
      def _():
        fire(bc + 3, rows[nb], sems[nb])

      drain(rows[b], sems[b])

      ob = b % 2

      @pl.when(bc >= 2)
      def _():
        pltpu.make_async_copy(outs[ob], out_slice(bc - 2), osems[ob]).wait()

      combine(rows[b], outs[ob])
      pltpu.async_copy(outs[ob], out_slice(bc), osems[ob])
    return 0

  lax.fori_loop(0, NCHUNK // 4, chunk_pair, 0)

  for b in range(2):
    pltpu.make_async_copy(outs[b], out_slice(NCHUNK - 2 + b), osems[b]).wait()


@jax.jit
def kernel(indices, table):
  idx3 = indices.reshape(NW, IDX_ROWS, IDX_MINOR)
  mesh = plsc.VectorSubcoreMesh(core_axis_name="c", subcore_axis_name="s")
  f = pl.kernel(
      _body,
      out_type=jax.ShapeDtypeStruct((BATCH, EMBED_DIM), jnp.float32),
      mesh=mesh,
      scratch_types=[
          pltpu.VMEM((IDX_ROWS, IDX_MINOR), jnp.int32),
          pltpu.VMEM((CHUNK_ROWS, EMBED_DIM), jnp.float32),
          pltpu.VMEM((CHUNK_ROWS, EMBED_DIM), jnp.float32),
          pltpu.VMEM((CHUNK_ROWS, EMBED_DIM), jnp.float32),
          pltpu.VMEM((CHUNK_ROWS, EMBED_DIM), jnp.float32),
          pltpu.VMEM((CHUNK_SAMPLES, EMBED_DIM), jnp.float32),
          pltpu.VMEM((CHUNK_SAMPLES, EMBED_DIM), jnp.float32),
          pltpu.SemaphoreType.DMA,
          pltpu.SemaphoreType.DMA,
          pltpu.SemaphoreType.DMA,
          pltpu.SemaphoreType.DMA,
          pltpu.SemaphoreType.DMA,
          pltpu.SemaphoreType.DMA,
      ],
  )
  return f(idx3, table)


# speedup vs baseline: 1.0066x; 1.0066x over previous
import jax
import jax.numpy as jnp
from jax import lax
from jax.experimental import pallas as pl
from jax.experimental.pallas import tpu as pltpu
from jax.experimental.pallas import tpu_sc as plsc

BATCH = 16384
SEQ = 20
EMBED_DIM = 32
NC = 2
NS = 16
NW = NC * NS

B_PER_W = BATCH // NW
IDX_PER_W = B_PER_W * SEQ
IDX_MINOR = 128
IDX_ROWS = IDX_PER_W // IDX_MINOR
CHUNK_SAMPLES = 16
CHUNK_ROWS = CHUNK_SAMPLES * SEQ
GROUPS_PER_CHUNK = CHUNK_ROWS // 16
NCHUNK = B_PER_W // CHUNK_SAMPLES
INV_SEQ = 1.0 / SEQ


def _tree_sum(vals):
  while len(vals) > 1:
    nxt = [vals[i] + vals[i + 1] for i in range(0, len(vals) - 1, 2)]
    if len(vals) % 2:
      nxt.append(vals[-1])
    vals = nxt
  return vals[0]


def _body(idx_hbm, table_hbm, out_hbm,
          idx_v, rows0, rows1, out0, out1, sem0, sem1, osem0, osem1):
  wid = lax.axis_index("s") * NC + lax.axis_index("c")
  base = wid * B_PER_W

  pltpu.sync_copy(idx_hbm.at[wid], idx_v)

  rows = (rows0, rows1)
  sems = (sem0, sem1)
  outs = (out0, out1)
  osems = (osem0, osem1)

  def fire(bc, buf, sem):
    def grp(g, _):
      gg = bc * GROUPS_PER_CHUNK + g
      vec = idx_v[gg >> 3, pl.ds((gg & 7) * 16, 16)]
      for lane in range(16):
        pltpu.async_copy(
            table_hbm.at[vec[lane]],
            buf.at[g * 16 + lane],
            sem,
        )
      return 0
    lax.fori_loop(0, GROUPS_PER_CHUNK, grp, 0)

  def drain(buf, sem):
    def grp(g, _):
      pltpu.make_async_copy(
          table_hbm.at[pl.ds(0, 16)],
          buf.at[pl.ds(g * 16, 16)],
          sem,
      ).wait()
      return 0
    lax.fori_loop(0, GROUPS_PER_CHUNK, grp, 0)

  def combine(buf, out_v):
    @plsc.parallel_loop(0, CHUNK_SAMPLES, unroll=2)
    def _(i):
      r0 = i * SEQ
      lo = _tree_sum([buf[r0 + s, pl.ds(0, 16)] for s in range(SEQ)])
      hi = _tree_sum([buf[r0 + s, pl.ds(16, 16)] for s in range(SEQ)])
      out_v[i, pl.ds(0, 16)] = lo * INV_SEQ
      out_v[i, pl.ds(16, 16)] = hi * INV_SEQ

  def out_slice(bc):
    return out_hbm.at[pl.ds(base + bc * CHUNK_SAMPLES, CHUNK_SAMPLES)]

  fire(0, rows[0], sems[0])

  def chunk_pair(g, _):
    for b in range(2):
      bc = 2 * g + b
      nb = 1 - b

      @pl.when(bc + 1 < NCHUNK)
      def _():
        fire(bc + 1, rows[nb], sems[nb])

      drain(rows[b], sems[b])

      @pl.when(bc >= 2)
      def _():
        pltpu.make_async_copy(outs[b], out_slice(bc - 2), osems[b]).wait()

      combine(rows[b], outs[b])
      pltpu.async_copy(outs[b], out_slice(bc), osems[b])
    return 0

  lax.fori_loop(0, NCHUNK // 2, chunk_pair, 0)

  for b in range(2):
    pltpu.make_async_copy(outs[b], out_slice(NCHUNK - 2 + b), osems[b]).wait()


@jax.jit
def kernel(indices, table):
  idx3 = indices.reshape(NW, IDX_ROWS, IDX_MINOR)
  mesh = plsc.VectorSubcoreMesh(core_axis_name="c", subcore_axis_name="s")
  f = pl.kernel(
      _body,
      out_type=jax.ShapeDtypeStruct((BATCH, EMBED_DIM), jnp.float32),
      mesh=mesh,
      scratch_types=[
          pltpu.VMEM((IDX_ROWS, IDX_MINOR), jnp.int32),
          pltpu.VMEM((CHUNK_ROWS, EMBED_DIM), jnp.float32),
          pltpu.VMEM((CHUNK_ROWS, EMBED_DIM), jnp.float32),
          pltpu.VMEM((CHUNK_SAMPLES, EMBED_DIM), jnp.float32),
          pltpu.VMEM((CHUNK_SAMPLES, EMBED_DIM), jnp.float32),
          pltpu.SemaphoreType.DMA,
          pltpu.SemaphoreType.DMA,
          pltpu.SemaphoreType.DMA,
          pltpu.SemaphoreType.DMA,
      ],
  )
  return f(idx3, table)
